# baseline (speedup 1.0000x reference)
import jax
import jax.numpy as jnp
from jax import lax
from jax.experimental import pallas as pl
from jax.experimental.pallas import tpu as pltpu

_BN_EPS = 1e-5
_LOG2E = 1.4426950408889634
_TILES = (1024, 3072, 4096, 4096, 2048, 1024, 1024)
_N_ROWS_OUT = 4


def _prodlda_body(x_ref, w_hbm, o_hbm, wbuf, ebuf, obuf, l_ref, sem_w, sem_o):
    B = x_ref.shape[0]
    Br = B // _N_ROWS_OUT
    offs = [0]
    for sz in _TILES:
        offs.append(offs[-1] + sz)

    def w_copy(i):
        cols = slice(offs[i], offs[i + 1])
        return pltpu.make_async_copy(
            w_hbm.at[:, cols], wbuf.at[:, cols], sem_w.at[i])

    def o_copy(r, s):
        rows = slice(r * Br, (r + 1) * Br)
        return pltpu.make_async_copy(
            obuf.at[s], o_hbm.at[rows, :], sem_o.at[r])

    for i in range(len(_TILES)):
        w_copy(i).start()

    l = jnp.zeros((B, 1), jnp.float32)
    for i in range(len(_TILES)):
        cols = slice(offs[i], offs[i + 1])
        w_copy(i).wait()
        t32 = jnp.dot(x_ref[...], wbuf[:, cols],
                      preferred_element_type=jnp.float32)
        s1 = jnp.sum(t32, axis=0, keepdims=True)
        s2 = jnp.sum(t32 * t32, axis=0, keepdims=True)
        mu = s1 * (1.0 / B)
        var = s2 * (1.0 / B) - mu * mu
        a2 = lax.rsqrt(var + _BN_EPS) * _LOG2E
        b2 = -mu * a2
        e = jnp.exp2(t32 * a2 + b2)
        ebuf[:, cols] = e.astype(jnp.bfloat16)
        l = l + jnp.sum(e, axis=1, keepdims=True)
    l_ref[...] = l

    inv = 1.0 / l_ref[...]
    for r in range(_N_ROWS_OUT):
        rows = slice(r * Br, (r + 1) * Br)
        s = r % 2
        if r >= 2:
            o_copy(r - 2, r - 2).wait()
        obuf[s] = ebuf[rows, :].astype(jnp.float32) * inv[rows, :]
        o_copy(r, s).start()
    for r in range(max(0, _N_ROWS_OUT - 2), _N_ROWS_OUT):
        o_copy(r, r % 2).wait()


def kernel(x, beta_weight_t):
    B, K = x.shape
    K2, V = beta_weight_t.shape
    assert K == K2
    assert sum(_TILES) == V and B % _N_ROWS_OUT == 0

    cost = pl.CostEstimate(
        flops=2 * B * V * K,
        transcendentals=B * V,
        bytes_accessed=V * K * 4 + B * K * 4 + B * V * 4,
    )

    return pl.pallas_call(
        _prodlda_body,
        out_shape=jax.ShapeDtypeStruct((B, V), jnp.float32),
        in_specs=[
            pl.BlockSpec(memory_space=pltpu.MemorySpace.VMEM),
            pl.BlockSpec(memory_space=pltpu.MemorySpace.HBM),
        ],
        out_specs=pl.BlockSpec(memory_space=pltpu.MemorySpace.HBM),
        scratch_shapes=[
            pltpu.VMEM((K, V), jnp.float32),
            pltpu.VMEM((B, V), jnp.bfloat16),
            pltpu.VMEM((2, B // _N_ROWS_OUT, V), jnp.float32),
            pltpu.VMEM((B, 1), jnp.float32),
            pltpu.SemaphoreType.DMA((len(_TILES),)),
            pltpu.SemaphoreType.DMA((_N_ROWS_OUT,)),
        ],
        compiler_params=pltpu.CompilerParams(
            vmem_limit_bytes=int(58 << 20),
        ),
        cost_estimate=cost,
    )(x, beta_weight_t)
